# baseline (speedup 1.0000x reference)
import functools

import jax
import jax.numpy as jnp
from jax import lax
from jax.experimental import pallas as pl
from jax.experimental.pallas import tpu as pltpu
from jax.experimental.pallas import tpu_sc as plsc

T = 2048
HIDDEN = 2048
QLORA = 1536
NH = 16
HD = 128
RD = 64
HALF = RD // 2
TOPK = 1024
FP8_MAX = 448.0
SS = HD ** -0.5
BT = 256

NW = 32
RPW = T // NW
RB = 8
NBATCH = RPW // RB


def _rope2d(x, cos, sin):
    x1 = x[:, :HALF]
    x2 = x[:, HALF:RD]
    return jnp.concatenate(
        [x1 * cos - x2 * sin, x1 * sin + x2 * cos, x[:, RD:]], axis=1)


def _rowred(x):
    rows = x.shape[0]
    y = x.reshape(rows, 16, 8)
    acc = y[:, 0]
    for v in range(1, 16):
        acc = acc + y[:, v]
    while acc.shape[-1] > 1:
        h = acc.shape[-1] // 2
        acc = acc[:, :h] + acc[:, h:]
    return acc


def _prep_kernel(hid_ref, wk_ref, ww_ref, bw_ref, g_ref, b_ref, cos_ref,
                 sin_ref, k_out, w_out):
    h = hid_ref[...]
    kt = jax.lax.dot_general(wk_ref[...], h, (((0,), (1,)), ((), ())),
                             preferred_element_type=jnp.float32)
    k = kt.T
    mu = _rowred(k) * (1.0 / HD)
    var = _rowred((k - mu) ** 2) * (1.0 / HD)
    k = (k - mu) / jnp.sqrt(var + 1e-6) * g_ref[...] + b_ref[...]
    k_out[...] = _rope2d(k, cos_ref[...], sin_ref[...])
    wt = jax.lax.dot_general(ww_ref[...], h, (((0,), (1,)), ((), ())),
                             preferred_element_type=jnp.float32)
    w_out[...] = (wt.T + bw_ref[...]) * SS * (NH ** -0.5)


def _ceil_log2_exp2_neg(x):
    b = pltpu.bitcast(x, jnp.int32)
    exp = ((b >> 23) & 0xFF) - 127
    mant = b & 0x7FFFFF
    e = exp + jnp.where(mant != 0, 1, 0)
    return pltpu.bitcast(((-e) + 127) << 23, jnp.float32)


def _score_kernel(qr_ref, wqb_ref, k_ref, w_ref, cos_ref, sin_ref, out_ref):
    tb = pl.program_id(0)
    q = jax.lax.dot_general(qr_ref[...], wqb_ref[...], (((1,), (0,)), ((), ())),
                            preferred_element_type=jnp.float32)
    k = k_ref[...]
    w = w_ref[...]
    cos = cos_ref[...]
    sin = sin_ref[...]
    acc = jnp.zeros((BT, T), jnp.float32)
    for h in range(NH):
        qh = _rope2d(q[:, h * HD:(h + 1) * HD], cos, sin)
        amax = jnp.maximum(jnp.max(jnp.abs(qh), axis=-1, keepdims=True), 1e-4)
        sinv = _ceil_log2_exp2_neg(amax / FP8_MAX)
        qs = jnp.clip(qh * sinv, -FP8_MAX, FP8_MAX)
        logits = jax.lax.dot_general(qs, k, (((1,), (1,)), ((), ())),
                                     preferred_element_type=jnp.float32)
        wh = w[:, h:h + 1] / sinv
        acc = acc + wh * jax.nn.relu(logits)
    row = tb * BT + jax.lax.broadcasted_iota(jnp.int32, (BT, T), 0)
    col = jax.lax.broadcasted_iota(jnp.int32, (BT, T), 1)
    s = jnp.where(col <= row, acc, -1e30)
    b = pltpu.bitcast(s, jnp.int32)
    out_ref[...] = jnp.where(b < 0, b, jnp.bitwise_and(jnp.bitwise_not(b),
                                                       0x7FFFFFFF))


def _scores_keys(hidden_states, qr, positions, Wq_b, Wk, ln_gamma, ln_beta,
                 Ww, bw):
    inv = 1.0 / (10000.0 ** (jnp.arange(HALF, dtype=jnp.float32) / HALF))
    ang = positions.astype(jnp.float32)[:, None] * inv[None, :]
    cos = jnp.cos(ang)
    sin = jnp.sin(ang)

    k_rope, w_fold = pl.pallas_call(
        _prep_kernel,
        out_shape=(
            jax.ShapeDtypeStruct((T, HD), jnp.float32),
            jax.ShapeDtypeStruct((T, NH), jnp.float32),
        ),
    )(hidden_states, Wk, Ww, bw.reshape(1, NH), ln_gamma.reshape(1, HD),
      ln_beta.reshape(1, HD), cos, sin)

    grid = (T // BT,)
    keys = pl.pallas_call(
        _score_kernel,
        grid=grid,
        in_specs=[
            pl.BlockSpec((BT, QLORA), lambda i: (i, 0)),
            pl.BlockSpec((QLORA, NH * HD), lambda i: (0, 0)),
            pl.BlockSpec((T, HD), lambda i: (0, 0)),
            pl.BlockSpec((BT, NH), lambda i: (i, 0)),
            pl.BlockSpec((BT, HALF), lambda i: (i, 0)),
            pl.BlockSpec((BT, HALF), lambda i: (i, 0)),
        ],
        out_specs=pl.BlockSpec((BT, T), lambda i: (i, 0)),
        out_shape=jax.ShapeDtypeStruct((T, T), jnp.int32),
    )(qr, Wq_b, k_rope, w_fold, cos, sin)
    return keys


_sc_mesh = plsc.VectorSubcoreMesh(core_axis_name="c", subcore_axis_name="s")


@functools.partial(
    pl.kernel,
    out_type=(jax.ShapeDtypeStruct((T * TOPK,), jnp.float32),
              jax.ShapeDtypeStruct((T * TOPK,), jnp.int32)),
    mesh=_sc_mesh,
    compiler_params=pltpu.CompilerParams(needs_layout_passes=False),
    scratch_types=[pltpu.VMEM((RB * T,), jnp.int32),
                   pltpu.VMEM((RB * T,), jnp.int32),
                   pltpu.VMEM((RB * T,), jnp.int32),
                   pltpu.VMEM((RB * T,), jnp.int32),
                   pltpu.VMEM((16 * 256,), jnp.int32),
                   pltpu.VMEM((16 * 256,), jnp.int32),
                   pltpu.VMEM((RB * TOPK,), jnp.float32)],
)
def _sc_topk(keys_hbm, vals_hbm, idx_hbm, kv_a, kv_b, ix_a, ix_b, hist, cur,
             vbuf):
    wid = lax.axis_index("s") * 2 + lax.axis_index("c")
    lanes = lax.iota(jnp.int32, 16)
    rp = lanes >> 1
    halfsel = lanes & 1
    src_base = rp * T + halfsel * TOPK
    hbase = lanes * 256
    ones = jnp.full((16,), 1, jnp.int32)

    def batch_body(bb, carry0):
        row0 = wid * RPW + bb * RB

        def dma_in(r, c):
            pltpu.sync_copy(keys_hbm.at[pl.ds((row0 + r) * T, T)],
                            kv_a.at[pl.ds(r * T, T)])
            return c

        lax.fori_loop(0, RB, dma_in, 0)

        for p, (src, dst, isrc, idst) in enumerate(
                [(kv_a, kv_b, None, ix_b), (kv_b, kv_a, ix_b, ix_a),
                 (kv_a, kv_b, ix_a, ix_b), (kv_b, kv_a, ix_b, ix_a)]):
            sh = 8 * p

            def zero_body(i, c):
                hist[pl.ds(i * 16, 16)] = jnp.zeros((16,), jnp.int32)
                return c

            lax.fori_loop(0, 256, zero_body, 0)

            def hist_body(i, c):
                x = plsc.load_gather(src, [src_base + i])
                d = (x >> sh) & 255
                plsc.addupdate_scatter(hist, [hbase + d], ones)
                return c

            lax.fori_loop(0, TOPK, hist_body, 0, unroll=2)

            for rr in range(RB):
                e_base = (2 * rr) * 256
                o_base = (2 * rr + 1) * 256

                def scan_body(cix, carry, e_base=e_base, o_base=o_base):
                    h0 = hist[pl.ds(e_base + cix * 16, 16)]
                    h1 = hist[pl.ds(o_base + cix * 16, 16)]
                    t = h0 + h1
                    s_ = plsc.cumsum(t)
                    excl = s_ - t + carry
                    cur[pl.ds(e_base + cix * 16, 16)] = excl
                    cur[pl.ds(o_base + cix * 16, 16)] = excl + h0
                    return carry + jnp.sum(t)

                lax.fori_loop(0, 16, scan_body, 0)

            if p == 0:
                def place_body(i, c):
                    x = plsc.load_gather(src, [src_base + i])
                    d = (x >> sh) & 255
                    pos = plsc.load_gather(cur, [hbase + d])
                    dest = rp * T + pos
                    plsc.store_scatter(dst, [dest], x)
                    plsc.store_scatter(idst, [dest], halfsel * TOPK + i)
                    plsc.addupdate_scatter(cur, [hbase + d], ones)
                    return c
            else:
                def place_body(i, c, src=src, isrc=isrc, dst=dst, idst=idst,
                               sh=sh):
                    x = plsc.load_gather(src, [src_base + i])
                    d = (x >> sh) & 255
                    pos = plsc.load_gather(cur, [hbase + d])
                    dest = rp * T + pos
                    plsc.store_scatter(dst, [dest], x)
                    ix = plsc.load_gather(isrc, [src_base + i])
                    plsc.store_scatter(idst, [dest], ix)
                    plsc.addupdate_scatter(cur, [hbase + d], ones)
                    return c

            lax.fori_loop(0, TOPK, place_body, 0, unroll=2)

        for rr in range(RB):
            def conv_body(i, c, rr=rr):
                x = kv_a[pl.ds(rr * T + i * 16, 16)]
                b = jnp.where(x < 0, x,
                              jnp.bitwise_and(jnp.bitwise_not(x), 0x7FFFFFFF))
                vbuf[pl.ds(rr * TOPK + i * 16, 16)] = plsc.bitcast(
                    b, jnp.float32)
                return c

            lax.fori_loop(0, TOPK // 16, conv_body, 0, unroll=2)
            pltpu.sync_copy(vbuf.at[pl.ds(rr * TOPK, TOPK)],
                            vals_hbm.at[pl.ds((row0 + rr) * TOPK, TOPK)])
            pltpu.sync_copy(ix_a.at[pl.ds(rr * T, TOPK)],
                            idx_hbm.at[pl.ds((row0 + rr) * TOPK, TOPK)])
        return carry0

    lax.fori_loop(0, NBATCH, batch_body, 0)


def kernel(hidden_states, qr, positions, Wq_b, Wk, ln_gamma, ln_beta, Ww, bw):
    keys = _scores_keys(hidden_states, qr, positions, Wq_b, Wk, ln_gamma,
                        ln_beta, Ww, bw)
    vals, idx = _sc_topk(keys.reshape(T * T))
    return vals.reshape(T, TOPK), idx.reshape(T, TOPK)


# speedup vs baseline: 1.0016x; 1.0016x over previous
import functools

import jax
import jax.numpy as jnp
from jax import lax
from jax.experimental import pallas as pl
from jax.experimental.pallas import tpu as pltpu
from jax.experimental.pallas import tpu_sc as plsc

T = 2048
HIDDEN = 2048
QLORA = 1536
NH = 16
HD = 128
RD = 64
HALF = RD // 2
TOPK = 1024
FP8_MAX = 448.0
SS = HD ** -0.5
BT = 256

NW = 32
RPW = T // NW
RB = 8
NBATCH = RPW // RB
NCH = 4
CL = TOPK // NCH


def _rope2d(x, cos, sin):
    x1 = x[:, :HALF]
    x2 = x[:, HALF:RD]
    return jnp.concatenate(
        [x1 * cos - x2 * sin, x1 * sin + x2 * cos, x[:, RD:]], axis=1)


def _rowred(x):
    rows = x.shape[0]
    y = x.reshape(rows, 16, 8)
    acc = y[:, 0]
    for v in range(1, 16):
        acc = acc + y[:, v]
    while acc.shape[-1] > 1:
        h = acc.shape[-1] // 2
        acc = acc[:, :h] + acc[:, h:]
    return acc


def _prep_kernel(hid_ref, wk_ref, ww_ref, bw_ref, g_ref, b_ref, cos_ref,
                 sin_ref, k_out, w_out):
    h = hid_ref[...]
    kt = jax.lax.dot_general(wk_ref[...], h, (((0,), (1,)), ((), ())),
                             preferred_element_type=jnp.float32)
    k = kt.T
    mu = _rowred(k) * (1.0 / HD)
    var = _rowred((k - mu) ** 2) * (1.0 / HD)
    k = (k - mu) / jnp.sqrt(var + 1e-6) * g_ref[...] + b_ref[...]
    k_out[...] = _rope2d(k, cos_ref[...], sin_ref[...])
    wt = jax.lax.dot_general(ww_ref[...], h, (((0,), (1,)), ((), ())),
                             preferred_element_type=jnp.float32)
    w_out[...] = (wt.T + bw_ref[...]) * SS * (NH ** -0.5)


def _ceil_log2_exp2_neg(x):
    b = pltpu.bitcast(x, jnp.int32)
    exp = ((b >> 23) & 0xFF) - 127
    mant = b & 0x7FFFFF
    e = exp + jnp.where(mant != 0, 1, 0)
    return pltpu.bitcast(((-e) + 127) << 23, jnp.float32)


def _score_kernel(qr_ref, wqb_ref, k_ref, w_ref, cos_ref, sin_ref, out_ref):
    tb = pl.program_id(0)
    q = jax.lax.dot_general(qr_ref[...], wqb_ref[...], (((1,), (0,)), ((), ())),
                            preferred_element_type=jnp.float32)
    k = k_ref[...]
    w = w_ref[...]
    cos = cos_ref[...]
    sin = sin_ref[...]
    acc = jnp.zeros((BT, T), jnp.float32)
    for h in range(NH):
        qh = _rope2d(q[:, h * HD:(h + 1) * HD], cos, sin)
        amax = jnp.maximum(jnp.max(jnp.abs(qh), axis=-1, keepdims=True), 1e-4)
        sinv = _ceil_log2_exp2_neg(amax / FP8_MAX)
        qs = jnp.clip(qh * sinv, -FP8_MAX, FP8_MAX)
        logits = jax.lax.dot_general(qs, k, (((1,), (1,)), ((), ())),
                                     preferred_element_type=jnp.float32)
        wh = w[:, h:h + 1] / sinv
        acc = acc + wh * jax.nn.relu(logits)
    row = tb * BT + jax.lax.broadcasted_iota(jnp.int32, (BT, T), 0)
    col = jax.lax.broadcasted_iota(jnp.int32, (BT, T), 1)
    s = jnp.where(col <= row, acc, -1e30)
    b = pltpu.bitcast(s, jnp.int32)
    out_ref[...] = jnp.where(b < 0, b, jnp.bitwise_and(jnp.bitwise_not(b),
                                                       0x7FFFFFFF))


def _scores_keys(hidden_states, qr, positions, Wq_b, Wk, ln_gamma, ln_beta,
                 Ww, bw):
    inv = 1.0 / (10000.0 ** (jnp.arange(HALF, dtype=jnp.float32) / HALF))
    ang = positions.astype(jnp.float32)[:, None] * inv[None, :]
    cos = jnp.cos(ang)
    sin = jnp.sin(ang)

    k_rope, w_fold = pl.pallas_call(
        _prep_kernel,
        out_shape=(
            jax.ShapeDtypeStruct((T, HD), jnp.float32),
            jax.ShapeDtypeStruct((T, NH), jnp.float32),
        ),
    )(hidden_states, Wk, Ww, bw.reshape(1, NH), ln_gamma.reshape(1, HD),
      ln_beta.reshape(1, HD), cos, sin)

    grid = (T // BT,)
    keys = pl.pallas_call(
        _score_kernel,
        grid=grid,
        in_specs=[
            pl.BlockSpec((BT, QLORA), lambda i: (i, 0)),
            pl.BlockSpec((QLORA, NH * HD), lambda i: (0, 0)),
            pl.BlockSpec((T, HD), lambda i: (0, 0)),
            pl.BlockSpec((BT, NH), lambda i: (i, 0)),
            pl.BlockSpec((BT, HALF), lambda i: (i, 0)),
            pl.BlockSpec((BT, HALF), lambda i: (i, 0)),
        ],
        out_specs=pl.BlockSpec((BT, T), lambda i: (i, 0)),
        out_shape=jax.ShapeDtypeStruct((T, T), jnp.int32),
    )(qr, Wq_b, k_rope, w_fold, cos, sin)
    return keys


_sc_mesh = plsc.VectorSubcoreMesh(core_axis_name="c", subcore_axis_name="s")


@functools.partial(
    pl.kernel,
    out_type=(jax.ShapeDtypeStruct((T * TOPK,), jnp.float32),
              jax.ShapeDtypeStruct((T * TOPK,), jnp.int32)),
    mesh=_sc_mesh,
    compiler_params=pltpu.CompilerParams(needs_layout_passes=False),
    scratch_types=[pltpu.VMEM((RB * T,), jnp.int32),
                   pltpu.VMEM((RB * T,), jnp.int32),
                   pltpu.VMEM((RB * T,), jnp.int32),
                   pltpu.VMEM((RB * T,), jnp.int32),
                   pltpu.VMEM((16 * 256,), jnp.int32),
                   pltpu.VMEM((16 * 256,), jnp.int32),
                   pltpu.VMEM((16 * 256,), jnp.int32),
                   pltpu.VMEM((16 * 256,), jnp.int32),
                   pltpu.VMEM((16 * 256,), jnp.int32),
                   pltpu.VMEM((16 * 256,), jnp.int32),
                   pltpu.VMEM((16 * 256,), jnp.int32),
                   pltpu.VMEM((16 * 256,), jnp.int32),
                   pltpu.VMEM((RB * TOPK,), jnp.float32)],
)
def _sc_topk(keys_hbm, vals_hbm, idx_hbm, kv_a, kv_b, ix_a, ix_b,
             h0, h1, h2, h3, c0, c1, c2, c3, vbuf):
    wid = lax.axis_index("s") * 2 + lax.axis_index("c")
    lanes = lax.iota(jnp.int32, 16)
    rp = lanes >> 1
    halfsel = lanes & 1
    seg0 = rp * T + halfsel * TOPK
    hbase = lanes * 256
    ones = jnp.full((16,), 1, jnp.int32)
    hists = (h0, h1, h2, h3)
    curs = (c0, c1, c2, c3)

    def batch_body(bb, carry0):
        row0 = wid * RPW + bb * RB

        def dma_in(r, c):
            pltpu.sync_copy(keys_hbm.at[pl.ds((row0 + r) * T, T)],
                            kv_a.at[pl.ds(r * T, T)])
            return c

        lax.fori_loop(0, RB, dma_in, 0)

        for p, (src, dst, isrc, idst) in enumerate(
                [(kv_a, kv_b, None, ix_b), (kv_b, kv_a, ix_b, ix_a),
                 (kv_a, kv_b, ix_a, ix_b), (kv_b, kv_a, ix_b, ix_a)]):
            sh = 8 * p

            def zero_body(i, c):
                z = jnp.zeros((16,), jnp.int32)
                h0[pl.ds(i * 16, 16)] = z
                h1[pl.ds(i * 16, 16)] = z
                h2[pl.ds(i * 16, 16)] = z
                h3[pl.ds(i * 16, 16)] = z
                return c

            lax.fori_loop(0, 256, zero_body, 0, unroll=2)

            def hist_body(i, c, src=src, sh=sh):
                for ch in range(NCH):
                    x = plsc.load_gather(src, [seg0 + ch * CL + i])
                    d = (x >> sh) & 255
                    plsc.addupdate_scatter(hists[ch], [hbase + d], ones)
                return c

            lax.fori_loop(0, CL, hist_body, 0, unroll=2)

            for rr in range(RB):
                eb = (2 * rr) * 256
                ob = (2 * rr + 1) * 256

                def scan_body(cix, carry, eb=eb, ob=ob):
                    he = [h[pl.ds(eb + cix * 16, 16)] for h in hists]
                    ho = [h[pl.ds(ob + cix * 16, 16)] for h in hists]
                    t = he[0] + he[1] + he[2] + he[3] \
                        + ho[0] + ho[1] + ho[2] + ho[3]
                    s_ = plsc.cumsum(t)
                    acc = s_ - t + carry
                    for ch in range(NCH):
                        curs[ch][pl.ds(eb + cix * 16, 16)] = acc
                        acc = acc + he[ch]
                    for ch in range(NCH):
                        curs[ch][pl.ds(ob + cix * 16, 16)] = acc
                        acc = acc + ho[ch]
                    return carry + jnp.sum(t)

                lax.fori_loop(0, 16, scan_body, 0)

            if p == 0:
                def place_body(i, c, src=src, dst=dst, idst=idst, sh=sh):
                    for ch in range(NCH):
                        x = plsc.load_gather(src, [seg0 + ch * CL + i])
                        d = (x >> sh) & 255
                        pos = plsc.load_gather(curs[ch], [hbase + d])
                        dest = rp * T + pos
                        plsc.store_scatter(dst, [dest], x)
                        plsc.store_scatter(
                            idst, [dest], halfsel * TOPK + ch * CL + i)
                        plsc.addupdate_scatter(curs[ch], [hbase + d], ones)
                    return c
            else:
                def place_body(i, c, src=src, isrc=isrc, dst=dst, idst=idst,
                               sh=sh):
                    for ch in range(NCH):
                        x = plsc.load_gather(src, [seg0 + ch * CL + i])
                        d = (x >> sh) & 255
                        pos = plsc.load_gather(curs[ch], [hbase + d])
                        dest = rp * T + pos
                        plsc.store_scatter(dst, [dest], x)
                        ix = plsc.load_gather(isrc, [seg0 + ch * CL + i])
                        plsc.store_scatter(idst, [dest], ix)
                        plsc.addupdate_scatter(curs[ch], [hbase + d], ones)
                    return c

            lax.fori_loop(0, CL, place_body, 0, unroll=2)

        for rr in range(RB):
            def conv_body(i, c, rr=rr):
                x = kv_a[pl.ds(rr * T + i * 16, 16)]
                b = jnp.where(x < 0, x,
                              jnp.bitwise_and(jnp.bitwise_not(x), 0x7FFFFFFF))
                vbuf[pl.ds(rr * TOPK + i * 16, 16)] = plsc.bitcast(
                    b, jnp.float32)
                return c

            lax.fori_loop(0, TOPK // 16, conv_body, 0, unroll=4)
            pltpu.sync_copy(vbuf.at[pl.ds(rr * TOPK, TOPK)],
                            vals_hbm.at[pl.ds((row0 + rr) * TOPK, TOPK)])
            pltpu.sync_copy(ix_a.at[pl.ds(rr * T, TOPK)],
                            idx_hbm.at[pl.ds((row0 + rr) * TOPK, TOPK)])
        return carry0

    lax.fori_loop(0, NBATCH, batch_body, 0)


def kernel(hidden_states, qr, positions, Wq_b, Wk, ln_gamma, ln_beta, Ww, bw):
    keys = _scores_keys(hidden_states, qr, positions, Wq_b, Wk, ln_gamma,
                        ln_beta, Ww, bw)
    vals, idx = _sc_topk(keys.reshape(T * T))
    return vals.reshape(T, TOPK), idx.reshape(T, TOPK)
